# Initial kernel scaffold; baseline (speedup 1.0000x reference)
import jax
import jax.numpy as jnp
from jax.experimental import pallas as pl


def kernel(log_probs, best_prev):
    raise NotImplementedError("write your pallas kernel here")


# speedup vs baseline: 1.9792x; 1.9792x over previous
import functools

import jax
import jax.numpy as jnp
from jax import lax
from jax.experimental import pallas as pl
from jax.experimental.pallas import tpu as pltpu
from jax.experimental.pallas import tpu_sc as plsc

N_BEAMS = 4
EOS_ID = 2
KILL = -1000000000.0
VOCAB = 100000
ROWS = 32
C = 400
VPC = C // 16
CPB = VOCAB // C
CP = 256
SLAB = 10000
SLABS_PER_BEAM = VOCAB // SLAB
NSLAB = N_BEAMS * SLABS_PER_BEAM
CHUNKS_PER_SLAB = SLAB // C
ROW_STRIDE = N_BEAMS * VOCAB
NEG = float("-inf")
BIG = 1 << 30

_mesh = plsc.VectorSubcoreMesh(
    core_axis_name="c", subcore_axis_name="s", num_cores=2, num_subcores=16
)


@functools.partial(
    pl.kernel,
    out_type=jax.ShapeDtypeStruct((ROWS, 16), jnp.float32),
    mesh=_mesh,
    compiler_params=pltpu.CompilerParams(needs_layout_passes=False),
    scratch_types=[
        pltpu.VMEM((SLAB,), jnp.float32),
        pltpu.VMEM((N_BEAMS * CP,), jnp.float32),
        pltpu.VMEM((16,), jnp.float32),
        pltpu.VMEM((8 * C,), jnp.float32),
        pltpu.VMEM((8 * C,), jnp.int32),
        pltpu.VMEM((16,), jnp.float32),
        pltpu.SemaphoreType.DMA,
    ],
)
def _beam_topk(lp_hbm, bp_hbm, out_hbm, slab_ref, m_ref, bp_ref, cand_ref,
               g_ref, orow_ref, sem):
    r = lax.axis_index("s") * 2 + lax.axis_index("c")
    lanes = lax.iota(jnp.int32, 16)

    @pl.loop(0, (N_BEAMS * CP) // 16)
    def _(i):
        m_ref[pl.ds(i * 16, 16)] = jnp.full((16,), NEG, jnp.float32)

    pltpu.sync_copy(bp_hbm.at[r], bp_ref)
    bp_vec = bp_ref[...]

    @pl.loop(0, NSLAB)
    def _(s):
        pltpu.sync_copy(lp_hbm.at[pl.ds(r * ROW_STRIDE + s * SLAB, SLAB)],
                        slab_ref)
        b = s // SLABS_PER_BEAM
        cbase = (s % SLABS_PER_BEAM) * CHUNKS_PER_SLAB

        @pl.loop(0, CHUNKS_PER_SLAB)
        def _(k):
            base = k * C
            v = slab_ref[pl.ds(base, 16)]
            for j in range(1, VPC):
                v = jnp.maximum(v, slab_ref[pl.ds(base + j * 16, 16)])
            cmax = jnp.max(v)
            pos = b * CP + cbase + k
            plsc.store_scatter(m_ref, [jnp.full((16,), pos, jnp.int32)],
                               jnp.full((16,), cmax, jnp.float32), mask=lanes == 0)

    for b in range(N_BEAMS):
        pb = bp_vec[b]

        @pl.loop(0, CP // 16)
        def _(i, _pb=pb, _b=b):
            off = _b * CP + i * 16
            m_ref[pl.ds(off, 16)] = m_ref[pl.ds(off, 16)] + _pb

    sel_pos = []
    for k in range(8):
        def scan_body(i, carry):
            bv, bi = carry
            val = m_ref[pl.ds(i * 16, 16)]
            idx = i * 16 + lanes
            better = (val > bv) | ((val == bv) & (idx < bi))
            return (jnp.where(better, val, bv), jnp.where(better, idx, bi))

        bv, bi = pl.loop(
            0, (N_BEAMS * CP) // 16,
            init_carry=(jnp.full((16,), NEG, jnp.float32),
                        jnp.full((16,), BIG, jnp.int32)),
        )(scan_body)
        m = jnp.max(bv)
        pos = jnp.min(jnp.where(bv == m, bi, BIG))
        sel_pos.append(pos)
        plsc.store_scatter(m_ref, [jnp.full((16,), pos, jnp.int32)],
                           jnp.full((16,), NEG, jnp.float32), mask=lanes == 0)

    copies = []
    for k in range(8):
        b_k = sel_pos[k] // CP
        c_k = sel_pos[k] % CP
        off = r * ROW_STRIDE + b_k * VOCAB + c_k * C
        copies.append(
            pltpu.async_copy(lp_hbm.at[pl.ds(off, C)],
                             cand_ref.at[pl.ds(k * C, C)], sem))
    for cp in copies:
        cp.wait()

    for k in range(8):
        b_k = sel_pos[k] // CP
        c_k = sel_pos[k] % CP
        gbase = b_k * VOCAB + c_k * C
        pbv = plsc.load_gather(bp_ref, [jnp.full((16,), b_k, jnp.int32)])

        @pl.loop(0, VPC)
        def _(j, _k=k, _gb=gbase, _pbv=pbv):
            o = _k * C + j * 16
            cand_ref[pl.ds(o, 16)] = cand_ref[pl.ds(o, 16)] + _pbv
            g_ref[pl.ds(o, 16)] = _gb + j * 16 + lanes

    winners = []
    for k in range(8):
        def fin_body(i, carry):
            bv, bg, bp_ = carry
            val = cand_ref[pl.ds(i * 16, 16)]
            g = g_ref[pl.ds(i * 16, 16)]
            fp = i * 16 + lanes
            better = (val > bv) | ((val == bv) & (g < bg))
            return (jnp.where(better, val, bv), jnp.where(better, g, bg),
                    jnp.where(better, fp, bp_))

        bv, bg, bpos = pl.loop(
            0, (8 * C) // 16,
            init_carry=(jnp.full((16,), NEG, jnp.float32),
                        jnp.full((16,), BIG, jnp.int32),
                        jnp.full((16,), BIG, jnp.int32)),
        )(fin_body)
        m = jnp.max(bv)
        gi = jnp.min(jnp.where(bv == m, bg, BIG))
        p = jnp.min(jnp.where((bv == m) & (bg == gi), bpos, BIG))
        winners.append((m, gi))
        plsc.store_scatter(cand_ref, [jnp.full((16,), p, jnp.int32)],
                           jnp.full((16,), NEG, jnp.float32), mask=lanes == 0)

    fvals = jnp.full((16,), NEG, jnp.float32)
    vocab_v = jnp.zeros((16,), jnp.int32)
    beam_v = jnp.zeros((16,), jnp.int32)
    for k in range(8):
        m, gi = winners[k]
        vocab_k = gi % VOCAB
        beam_k = gi // VOCAB
        mk = jnp.where(vocab_k == EOS_ID, KILL, m)
        fvals = jnp.where(lanes == k, mk, fvals)
        vocab_v = jnp.where(lanes == k, vocab_k, vocab_v)
        beam_v = jnp.where(lanes == k, beam_k, beam_v)

    out_f = jnp.zeros((16,), jnp.float32)
    out_i = jnp.zeros((16,), jnp.int32)
    for j in range(4):
        m = jnp.max(fvals)
        pos = jnp.min(jnp.where(fvals == m, lanes, BIG))
        onehot = lanes == pos
        vj = jnp.max(jnp.where(onehot, vocab_v, -1))
        bj = jnp.max(jnp.where(onehot, beam_v, -1))
        out_f = jnp.where(lanes == j, m, out_f)
        out_i = jnp.where(lanes == 4 + j, vj, out_i)
        out_i = jnp.where(lanes == 8 + j, bj, out_i)
        fvals = jnp.where(onehot, NEG, fvals)

    orow_ref[...] = jnp.where(lanes < 4, out_f,
                              plsc.bitcast(out_i, jnp.float32))
    pltpu.sync_copy(orow_ref, out_hbm.at[r])


def kernel(log_probs, best_prev):
    lp_flat = log_probs.reshape(-1)
    bp_pad = jnp.pad(best_prev, ((0, 0), (0, 16 - N_BEAMS)))
    out = _beam_topk(lp_flat, bp_pad)
    cont = out[:, 0:4]
    vocab = lax.bitcast_convert_type(out[:, 4:8], jnp.int32)
    beam = lax.bitcast_convert_type(out[:, 8:12], jnp.int32)
    return cont, vocab, beam, vocab.reshape(-1)


# speedup vs baseline: 2.3743x; 1.1996x over previous
import functools

import jax
import jax.numpy as jnp
from jax import lax
from jax.experimental import pallas as pl
from jax.experimental.pallas import tpu as pltpu
from jax.experimental.pallas import tpu_sc as plsc

N_BEAMS = 4
EOS_ID = 2
KILL = -1000000000.0
VOCAB = 100000
ROWS = 32
C = 400
VPC = C // 16
CPB = VOCAB // C
CP = 256
SLAB = 20000
SLABS_PER_BEAM = VOCAB // SLAB
NSLAB = N_BEAMS * SLABS_PER_BEAM
CHUNKS_PER_SLAB = SLAB // C
ROW_STRIDE = N_BEAMS * VOCAB
NEG = float("-inf")
BIG = 1 << 30

_mesh = plsc.VectorSubcoreMesh(
    core_axis_name="c", subcore_axis_name="s", num_cores=2, num_subcores=16
)


@functools.partial(
    pl.kernel,
    out_type=jax.ShapeDtypeStruct((ROWS, 16), jnp.float32),
    mesh=_mesh,
    compiler_params=pltpu.CompilerParams(needs_layout_passes=False),
    scratch_types=[
        pltpu.VMEM((SLAB,), jnp.float32),
        pltpu.VMEM((SLAB,), jnp.float32),
        pltpu.VMEM((N_BEAMS * CP,), jnp.float32),
        pltpu.VMEM((16,), jnp.float32),
        pltpu.VMEM((8 * C,), jnp.float32),
        pltpu.VMEM((8 * C,), jnp.int32),
        pltpu.VMEM((16,), jnp.float32),
        pltpu.SemaphoreType.DMA,
        pltpu.SemaphoreType.DMA,
        pltpu.SemaphoreType.DMA,
    ],
)
def _beam_topk(lp_hbm, bp_hbm, out_hbm, slab_a, slab_b, m_ref, bp_ref,
               cand_ref, g_ref, orow_ref, sem, sem_a, sem_b):
    r = lax.axis_index("s") * 2 + lax.axis_index("c")
    lanes = lax.iota(jnp.int32, 16)

    @pl.loop(0, (N_BEAMS * CP) // 16)
    def _(i):
        m_ref[pl.ds(i * 16, 16)] = jnp.full((16,), NEG, jnp.float32)

    pltpu.sync_copy(bp_hbm.at[r], bp_ref)
    bp_vec = bp_ref[...]

    def _src(s):
        return lp_hbm.at[pl.ds(r * ROW_STRIDE + s * SLAB, SLAB)]

    def _reduce_slab(buf, s):
        b = s // SLABS_PER_BEAM
        cbase = (s % SLABS_PER_BEAM) * CHUNKS_PER_SLAB

        @pl.loop(0, CHUNKS_PER_SLAB, unroll=2)
        def _(k):
            base = k * C
            v = buf[pl.ds(base, 16)]
            for j in range(1, VPC):
                v = jnp.maximum(v, buf[pl.ds(base + j * 16, 16)])
            cmax = jnp.max(v)
            pos = b * CP + cbase + k
            plsc.store_scatter(m_ref, [jnp.full((16,), pos, jnp.int32)],
                               jnp.full((16,), cmax, jnp.float32),
                               mask=lanes == 0)

    pltpu.async_copy(_src(0), slab_a, sem_a)

    @pl.loop(0, NSLAB // 2)
    def _(t):
        s0 = 2 * t
        s1 = s0 + 1
        pltpu.async_copy(_src(s1), slab_b, sem_b)
        pltpu.make_async_copy(_src(0), slab_a, sem_a).wait()
        _reduce_slab(slab_a, s0)
        s2 = jnp.minimum(s1 + 1, NSLAB - 1)
        pltpu.async_copy(_src(s2), slab_a, sem_a)
        pltpu.make_async_copy(_src(0), slab_b, sem_b).wait()
        _reduce_slab(slab_b, s1)

    pltpu.make_async_copy(_src(0), slab_a, sem_a).wait()

    for b in range(N_BEAMS):
        pb = bp_vec[b]

        @pl.loop(0, CP // 16)
        def _(i, _pb=pb, _b=b):
            off = _b * CP + i * 16
            m_ref[pl.ds(off, 16)] = m_ref[pl.ds(off, 16)] + _pb

    sel_pos = []
    for k in range(8):
        def scan_body(i, carry):
            bv, bi = carry
            val = m_ref[pl.ds(i * 16, 16)]
            idx = i * 16 + lanes
            better = (val > bv) | ((val == bv) & (idx < bi))
            return (jnp.where(better, val, bv), jnp.where(better, idx, bi))

        bv, bi = pl.loop(
            0, (N_BEAMS * CP) // 16,
            init_carry=(jnp.full((16,), NEG, jnp.float32),
                        jnp.full((16,), BIG, jnp.int32)),
        )(scan_body)
        m = jnp.max(bv)
        pos = jnp.min(jnp.where(bv == m, bi, BIG))
        sel_pos.append(pos)
        plsc.store_scatter(m_ref, [jnp.full((16,), pos, jnp.int32)],
                           jnp.full((16,), NEG, jnp.float32), mask=lanes == 0)

    copies = []
    for k in range(8):
        b_k = sel_pos[k] // CP
        c_k = sel_pos[k] % CP
        off = r * ROW_STRIDE + b_k * VOCAB + c_k * C
        copies.append(
            pltpu.async_copy(lp_hbm.at[pl.ds(off, C)],
                             cand_ref.at[pl.ds(k * C, C)], sem))
    for cp in copies:
        cp.wait()

    for k in range(8):
        b_k = sel_pos[k] // CP
        c_k = sel_pos[k] % CP
        gbase = b_k * VOCAB + c_k * C
        pbv = plsc.load_gather(bp_ref, [jnp.full((16,), b_k, jnp.int32)])

        @pl.loop(0, VPC)
        def _(j, _k=k, _gb=gbase, _pbv=pbv):
            o = _k * C + j * 16
            cand_ref[pl.ds(o, 16)] = cand_ref[pl.ds(o, 16)] + _pbv
            g_ref[pl.ds(o, 16)] = _gb + j * 16 + lanes

    winners = []
    for k in range(8):
        def fin_body(i, carry):
            bv, bg, bp_ = carry
            val = cand_ref[pl.ds(i * 16, 16)]
            g = g_ref[pl.ds(i * 16, 16)]
            fp = i * 16 + lanes
            better = (val > bv) | ((val == bv) & (g < bg))
            return (jnp.where(better, val, bv), jnp.where(better, g, bg),
                    jnp.where(better, fp, bp_))

        bv, bg, bpos = pl.loop(
            0, (8 * C) // 16,
            init_carry=(jnp.full((16,), NEG, jnp.float32),
                        jnp.full((16,), BIG, jnp.int32),
                        jnp.full((16,), BIG, jnp.int32)),
        )(fin_body)
        m = jnp.max(bv)
        gi = jnp.min(jnp.where(bv == m, bg, BIG))
        p = jnp.min(jnp.where((bv == m) & (bg == gi), bpos, BIG))
        winners.append((m, gi))
        plsc.store_scatter(cand_ref, [jnp.full((16,), p, jnp.int32)],
                           jnp.full((16,), NEG, jnp.float32), mask=lanes == 0)

    fvals = jnp.full((16,), NEG, jnp.float32)
    vocab_v = jnp.zeros((16,), jnp.int32)
    beam_v = jnp.zeros((16,), jnp.int32)
    for k in range(8):
        m, gi = winners[k]
        vocab_k = gi % VOCAB
        beam_k = gi // VOCAB
        mk = jnp.where(vocab_k == EOS_ID, KILL, m)
        fvals = jnp.where(lanes == k, mk, fvals)
        vocab_v = jnp.where(lanes == k, vocab_k, vocab_v)
        beam_v = jnp.where(lanes == k, beam_k, beam_v)

    out_f = jnp.zeros((16,), jnp.float32)
    out_i = jnp.zeros((16,), jnp.int32)
    for j in range(4):
        m = jnp.max(fvals)
        pos = jnp.min(jnp.where(fvals == m, lanes, BIG))
        onehot = lanes == pos
        vj = jnp.max(jnp.where(onehot, vocab_v, -1))
        bj = jnp.max(jnp.where(onehot, beam_v, -1))
        out_f = jnp.where(lanes == j, m, out_f)
        out_i = jnp.where(lanes == 4 + j, vj, out_i)
        out_i = jnp.where(lanes == 8 + j, bj, out_i)
        fvals = jnp.where(onehot, NEG, fvals)

    orow_ref[...] = jnp.where(lanes < 4, out_f,
                              plsc.bitcast(out_i, jnp.float32))
    pltpu.sync_copy(orow_ref, out_hbm.at[r])


def kernel(log_probs, best_prev):
    lp_flat = log_probs.reshape(-1)
    bp_pad = jnp.pad(best_prev, ((0, 0), (0, 16 - N_BEAMS)))
    out = _beam_topk(lp_flat, bp_pad)
    cont = out[:, 0:4]
    vocab = lax.bitcast_convert_type(out[:, 4:8], jnp.int32)
    beam = lax.bitcast_convert_type(out[:, 8:12], jnp.int32)
    return cont, vocab, beam, vocab.reshape(-1)


# speedup vs baseline: 3.3836x; 1.4251x over previous
import functools

import jax
import jax.numpy as jnp
from jax import lax
from jax.experimental import pallas as pl
from jax.experimental.pallas import tpu as pltpu
from jax.experimental.pallas import tpu_sc as plsc

N_BEAMS = 4
EOS_ID = 2
KILL = -1000000000.0
VOCAB = 100000
ROWS = 32
LANE = 128
TILES = 782
ALLOC_COLS = TILES * LANE
C = 512
CHB = 98
CP = 2 * CHB + 12
HCOLS = 392 * LANE
FC = 640
WMAX = 776 * LANE
FULL_COLS = 781 * LANE
TAILB = 8 * FC
CAND_N = TAILB + 128
NSLAB = 24
SLAB_T = 16
SLAB_C = SLAB_T * LANE
TPB = 104
NEG = float("-inf")
BIG = 1 << 30

_mesh = plsc.VectorSubcoreMesh(
    core_axis_name="c", subcore_axis_name="s", num_cores=2, num_subcores=16
)


@functools.partial(
    pl.kernel,
    out_type=jax.ShapeDtypeStruct((ROWS, 16), jnp.float32),
    mesh=_mesh,
    compiler_params=pltpu.CompilerParams(needs_layout_passes=False),
    scratch_types=[
        pltpu.VMEM((8, SLAB_C), jnp.float32),
        pltpu.VMEM((8, SLAB_C), jnp.float32),
        pltpu.VMEM((8, 1024), jnp.float32),
        pltpu.VMEM((8, FC), jnp.float32),
        pltpu.VMEM((8 * TPB,), jnp.float32),
        pltpu.VMEM((N_BEAMS * CP,), jnp.float32),
        pltpu.VMEM((16,), jnp.float32),
        pltpu.VMEM((8, 8, FC), jnp.float32),
        pltpu.VMEM((CAND_N,), jnp.float32),
        pltpu.VMEM((CAND_N,), jnp.int32),
        pltpu.VMEM((16,), jnp.float32),
        pltpu.VMEM((8, LANE), jnp.float32),
        pltpu.VMEM_SHARED((16 * 8 * TPB,), jnp.float32),
        pltpu.SemaphoreType.DMA,
        pltpu.SemaphoreType.DMA,
        pltpu.SemaphoreType.DMA,
    ],
)
def _beam_topk(lp_hbm, tail_hbm, bp_hbm, out_hbm, slab_a, slab_b, ep0, ep1,
               t_ref, m_ref, bp_ref, fetch_ref, cand_ref, g_ref, orow_ref,
               tail_ref, sh_ref, sem, sem_a, sem_b):
    cid = lax.axis_index("c")
    sid = lax.axis_index("s")
    r = cid * 16 + sid
    grow = (r // 2) * 8
    h = sid % 2
    rrho = 4 * (sid % 2)
    lanes = lax.iota(jnp.int32, 16)

    pltpu.sync_copy(bp_hbm.at[r], bp_ref)
    bp_vec = bp_ref[...]

    @pl.loop(0, (8 * TPB) // 16)
    def _(i):
        t_ref[pl.ds(i * 16, 16)] = jnp.full((16,), NEG, jnp.float32)

    col0 = HCOLS * h

    def _src(slab):
        return lp_hbm.at[pl.ds(grow, 8), pl.ds(col0 + slab * SLAB_C, SLAB_C)]

    def _reduce_slab(buf, slab):
        @pl.loop(0, SLAB_T // 4)
        def _(q):
            base = q * C
            for rho in range(8):
                v = buf[rho, pl.ds(base, 16)]
                for j in range(1, C // 16):
                    v = jnp.maximum(v, buf[rho, pl.ds(base + j * 16, 16)])
                cmax = jnp.max(v)
                pos = rho * TPB + slab * 4 + q
                plsc.store_scatter(t_ref, [jnp.full((16,), pos, jnp.int32)],
                                   jnp.full((16,), cmax, jnp.float32),
                                   mask=lanes == 0)

    pltpu.async_copy(_src(0), slab_a, sem_a)

    @pl.loop(0, NSLAB // 2)
    def _(t):
        s0 = 2 * t
        s1 = s0 + 1
        pltpu.async_copy(_src(s1), slab_b, sem_b)
        pltpu.make_async_copy(_src(0), slab_a, sem_a).wait()
        _reduce_slab(slab_a, s0)
        s2 = jnp.minimum(s1 + 1, NSLAB - 1)
        pltpu.async_copy(_src(s2), slab_a, sem_a)
        pltpu.make_async_copy(_src(0), slab_b, sem_b).wait()
        _reduce_slab(slab_b, s1)

    pltpu.make_async_copy(_src(0), slab_a, sem_a).wait()

    @pl.when(h == 0)
    def _():
        pltpu.sync_copy(lp_hbm.at[pl.ds(grow, 8), pl.ds(384 * LANE, 1024)],
                        ep0)
        for q in range(2):
            for rho in range(8):
                v = ep0[rho, pl.ds(q * C, 16)]
                for j in range(1, C // 16):
                    v = jnp.maximum(v, ep0[rho, pl.ds(q * C + j * 16, 16)])
                cmax = jnp.max(v)
                pos = rho * TPB + 96 + q
                plsc.store_scatter(t_ref, [jnp.full((16,), pos, jnp.int32)],
                                   jnp.full((16,), cmax, jnp.float32),
                                   mask=lanes == 0)

    @pl.when(h == 1)
    def _():
        pltpu.sync_copy(lp_hbm.at[pl.ds(grow, 8), pl.ds(776 * LANE, FC)],
                        ep1)
        for rho in range(8):
            v = ep1[rho, pl.ds(0, 16)]
            for j in range(1, 32):
                v = jnp.maximum(v, ep1[rho, pl.ds(j * 16, 16)])
            cmax = jnp.max(v)
            pos = rho * TPB + 96
            plsc.store_scatter(t_ref, [jnp.full((16,), pos, jnp.int32)],
                               jnp.full((16,), cmax, jnp.float32),
                               mask=lanes == 0)
            v = ep1[rho, pl.ds(C, 16)]
            for j in range(1, 8):
                v = jnp.maximum(v, ep1[rho, pl.ds(C + j * 16, 16)])
            cmax = jnp.max(v)
            pos = rho * TPB + 97
            plsc.store_scatter(t_ref, [jnp.full((16,), pos, jnp.int32)],
                               jnp.full((16,), cmax, jnp.float32),
                               mask=lanes == 0)

    pltpu.sync_copy(t_ref, sh_ref.at[pl.ds(sid * 8 * TPB, 8 * TPB)])
    plsc.subcore_barrier()
    p0 = 2 * (sid // 2)
    for b in range(N_BEAMS):
        for hh in range(2):
            pltpu.sync_copy(
                sh_ref.at[pl.ds((p0 + hh) * 8 * TPB + (rrho + b) * TPB, TPB)],
                m_ref.at[pl.ds(b * CP + hh * TPB, TPB)])

    for b in range(N_BEAMS):
        pb = bp_vec[b]

        @pl.loop(0, CP // 16)
        def _(i, _pb=pb, _b=b):
            off = _b * CP + i * 16
            m_ref[pl.ds(off, 16)] = m_ref[pl.ds(off, 16)] + _pb

    sel_pos = []
    for k in range(8):
        def scan_body(i, carry):
            bv, bi = carry
            val = m_ref[pl.ds(i * 16, 16)]
            idx = i * 16 + lanes
            better = (val > bv) | ((val == bv) & (idx < bi))
            return (jnp.where(better, val, bv), jnp.where(better, idx, bi))

        bv, bi = pl.loop(
            0, (N_BEAMS * CP) // 16,
            init_carry=(jnp.full((16,), NEG, jnp.float32),
                        jnp.full((16,), BIG, jnp.int32)),
        )(scan_body)
        m = jnp.max(bv)
        pos = jnp.min(jnp.where(bv == m, bi, BIG))
        sel_pos.append(pos)
        plsc.store_scatter(m_ref, [jnp.full((16,), pos, jnp.int32)],
                           jnp.full((16,), NEG, jnp.float32), mask=lanes == 0)

    chunk_info = []
    copies = []
    for k in range(8):
        pos = sel_pos[k]
        b_k = pos // CP
        q2 = pos % CP
        h_k = q2 // TPB
        cc_k = q2 % TPB
        chunk_col0 = HCOLS * h_k + C * cc_k
        fetch_col0 = jnp.minimum(chunk_col0, WMAX)

        chunk_info.append((b_k, chunk_col0, fetch_col0))
        copies.append(
            pltpu.async_copy(
                lp_hbm.at[pl.ds(grow, 8), pl.ds(fetch_col0, FC)],
                fetch_ref.at[k], sem))
    for cp in copies:
        cp.wait()

    for k in range(8):
        b_k, chunk_col0, fetch_col0 = chunk_info[k]
        pbv = plsc.load_gather(bp_ref, [jnp.full((16,), b_k, jnp.int32)])
        rho_k = rrho + b_k

        @pl.loop(0, FC // 16)
        def _(j, _k=k, _c0=chunk_col0, _f0=fetch_col0, _b=b_k, _pbv=pbv,
              _rho=rho_k):
            o = _k * FC + j * 16
            gcol = _f0 + j * 16 + lanes
            valid = (gcol >= _c0) & (gcol < _c0 + C)
            data = fetch_ref[_k, _rho, pl.ds(j * 16, 16)]
            cand_ref[pl.ds(o, 16)] = jnp.where(valid, data + _pbv, NEG)
            g_ref[pl.ds(o, 16)] = _b * VOCAB + gcol

    pltpu.sync_copy(tail_hbm.at[pl.ds(grow, 8)], tail_ref)
    for b in range(N_BEAMS):
        for v in range(2):
            data = plsc.load_gather(
                tail_ref, [jnp.full((16,), rrho + b, jnp.int32),
                           v * 16 + lanes])
            o = TAILB + b * 32 + v * 16
            cand_ref[pl.ds(o, 16)] = data + bp_vec[b]
            g_ref[pl.ds(o, 16)] = b * VOCAB + FULL_COLS + v * 16 + lanes

    winners = []
    for k in range(8):
        def fin_body(i, carry):
            bv, bg, bp_ = carry
            val = cand_ref[pl.ds(i * 16, 16)]
            g = g_ref[pl.ds(i * 16, 16)]
            fp = i * 16 + lanes
            better = (val > bv) | ((val == bv) & (g < bg))
            return (jnp.where(better, val, bv), jnp.where(better, g, bg),
                    jnp.where(better, fp, bp_))

        bv, bg, bpos = pl.loop(
            0, CAND_N // 16,
            init_carry=(jnp.full((16,), NEG, jnp.float32),
                        jnp.full((16,), BIG, jnp.int32),
                        jnp.full((16,), BIG, jnp.int32)),
        )(fin_body)
        m = jnp.max(bv)
        gi = jnp.min(jnp.where(bv == m, bg, BIG))
        p = jnp.min(jnp.where((bv == m) & (bg == gi), bpos, BIG))
        winners.append((m, gi))
        plsc.store_scatter(cand_ref, [jnp.full((16,), p, jnp.int32)],
                           jnp.full((16,), NEG, jnp.float32), mask=lanes == 0)

    fvals = jnp.full((16,), NEG, jnp.float32)
    vocab_v = jnp.zeros((16,), jnp.int32)
    beam_v = jnp.zeros((16,), jnp.int32)
    for k in range(8):
        m, gi = winners[k]
        vocab_k = gi % VOCAB
        beam_k = gi // VOCAB
        mk = jnp.where(vocab_k == EOS_ID, KILL, m)
        fvals = jnp.where(lanes == k, mk, fvals)
        vocab_v = jnp.where(lanes == k, vocab_k, vocab_v)
        beam_v = jnp.where(lanes == k, beam_k, beam_v)

    out_f = jnp.zeros((16,), jnp.float32)
    out_i = jnp.zeros((16,), jnp.int32)
    for j in range(4):
        m = jnp.max(fvals)
        pos = jnp.min(jnp.where(fvals == m, lanes, BIG))
        onehot = lanes == pos
        vj = jnp.max(jnp.where(onehot, vocab_v, -1))
        bj = jnp.max(jnp.where(onehot, beam_v, -1))
        out_f = jnp.where(lanes == j, m, out_f)
        out_i = jnp.where(lanes == 4 + j, vj, out_i)
        out_i = jnp.where(lanes == 8 + j, bj, out_i)
        fvals = jnp.where(onehot, NEG, fvals)

    orow_ref[...] = jnp.where(lanes < 4, out_f,
                              plsc.bitcast(out_i, jnp.float32))
    pltpu.sync_copy(orow_ref, out_hbm.at[r])


def kernel(log_probs, best_prev):
    bp_pad = jnp.pad(best_prev, ((0, 0), (0, 16 - N_BEAMS)))
    tail = jnp.pad(log_probs[:, FULL_COLS:], ((0, 0), (0, LANE - 32)))
    out = _beam_topk(log_probs, tail, bp_pad)
    cont = out[:, 0:4]
    vocab = lax.bitcast_convert_type(out[:, 4:8], jnp.int32)
    beam = lax.bitcast_convert_type(out[:, 8:12], jnp.int32)
    return cont, vocab, beam, vocab.reshape(-1)


# speedup vs baseline: 3.4664x; 1.0245x over previous
import functools

import jax
import jax.numpy as jnp
from jax import lax
from jax.experimental import pallas as pl
from jax.experimental.pallas import tpu as pltpu
from jax.experimental.pallas import tpu_sc as plsc

N_BEAMS = 4
EOS_ID = 2
KILL = -1000000000.0
VOCAB = 100000
ROWS = 32
LANES = 128
MW = 8
CW = 80
NMICRO = VOCAB // MW
NCHUNK = VOCAB // CW
CPAD = 1280
WIN = 6400
MPC = CW // MW
NEG = float("-inf")
BIG = 1 << 30

_mesh = plsc.VectorSubcoreMesh(
    core_axis_name="c", subcore_axis_name="s", num_cores=2, num_subcores=16
)


@functools.partial(
    pl.kernel,
    out_type=jax.ShapeDtypeStruct((ROWS, 16), jnp.float32),
    mesh=_mesh,
    compiler_params=pltpu.CompilerParams(needs_layout_passes=False),
    scratch_types=[
        pltpu.VMEM((CW, LANES), jnp.float32),
        pltpu.VMEM((CW, LANES), jnp.float32),
        pltpu.VMEM((MPC * 4 * 16,), jnp.float32),
        pltpu.VMEM((4 * CW * 16,), jnp.float32),
        pltpu.VMEM((16 * CW * 16,), jnp.float32),
        pltpu.VMEM((N_BEAMS * CPAD,), jnp.float32),
        pltpu.VMEM((8 * MPC * 4 * 16,), jnp.float32),
        pltpu.VMEM((8, MW, LANES), jnp.float32),
        pltpu.VMEM((16,), jnp.float32),
        pltpu.VMEM((16,), jnp.float32),
        pltpu.VMEM_SHARED((NMICRO * 4 * 16,), jnp.float32),
        pltpu.VMEM_SHARED((4 * 16 * CW * 16,), jnp.float32),
        pltpu.SemaphoreType.DMA,
        pltpu.SemaphoreType.DMA,
        pltpu.SemaphoreType.DMA,
    ],
)
def _beam_topk(lp_hbm, bp_hbm, out_hbm, blk_a, blk_b, mstage, cstage,
               plane_ref, m_ref, mg_ref, tile_ref, bp_ref, orow_ref,
               sh_micro, sh_chunk, sem, sem_a, sem_b):
    cid = lax.axis_index("c")
    sid = lax.axis_index("s")
    r = cid * 16 + sid
    lanes = lax.iota(jnp.int32, 16)

    pltpu.sync_copy(bp_hbm.at[r], bp_ref)
    bp_vec = bp_ref[...]

    win0 = WIN * sid
    lastc = jnp.where(sid == 15, (4000 // CW) - 1, (WIN // CW) - 1)
    lane0 = 64 * cid

    @pl.loop(0, (4 * CW * 16) // 16)
    def _(i):
        cstage[pl.ds(i * 16, 16)] = jnp.full((16,), NEG, jnp.float32)

    def _src(ci):
        off = pl.multiple_of(win0 + CW * ci, 8)
        return lp_hbm.at[pl.ds(off, CW), :]

    def _reduce_block(buf, ci):
        for jc in range(4):
            cvec = None
            for mi in range(MPC):
                v = buf[mi * MW, pl.ds(lane0 + 16 * jc, 16)]
                for p in range(1, MW):
                    v = jnp.maximum(v, buf[mi * MW + p,
                                           pl.ds(lane0 + 16 * jc, 16)])
                mstage[pl.ds((mi * 4 + jc) * 16, 16)] = v
                cvec = v if cvec is None else jnp.maximum(cvec, v)
            cstage[pl.ds((jc * CW + ci) * 16, 16)] = cvec
        m0 = (win0 + CW * ci) // MW
        pltpu.sync_copy(mstage, sh_micro.at[pl.ds(m0 * 64, MPC * 64)])

    cl0 = jnp.minimum(0, lastc)
    pltpu.async_copy(_src(cl0), blk_a, sem_a)

    @pl.loop(0, (WIN // CW) // 2)
    def _(t):
        c0 = jnp.minimum(2 * t, lastc)
        c1 = jnp.minimum(2 * t + 1, lastc)
        pltpu.async_copy(_src(c1), blk_b, sem_b)
        pltpu.make_async_copy(_src(cl0), blk_a, sem_a).wait()
        _reduce_block(blk_a, c0)
        c2 = jnp.minimum(2 * t + 2, lastc)
        pltpu.async_copy(_src(c2), blk_a, sem_a)
        pltpu.make_async_copy(_src(cl0), blk_b, sem_b).wait()
        _reduce_block(blk_b, c1)

    pltpu.make_async_copy(_src(cl0), blk_a, sem_a).wait()

    for jc in range(4):
        pltpu.sync_copy(
            cstage.at[pl.ds(jc * CW * 16, CW * 16)],
            sh_chunk.at[pl.ds((jc * 16 + sid) * CW * 16, CW * 16)])

    plsc.subcore_barrier()

    jloc = sid // 4
    lb0 = 4 * (sid % 4)
    pltpu.sync_copy(sh_chunk.at[pl.ds(jloc * 16 * CW * 16, 16 * CW * 16)],
                    plane_ref)

    for b in range(N_BEAMS):
        pb = bp_vec[b]

        @pl.loop(0, CPAD // 16)
        def _(gv, _b=b, _pb=pb):
            idx = (gv * 16 + lanes) * 16 + (lb0 + _b)
            vals = plsc.load_gather(plane_ref, [idx])
            m_ref[pl.ds(_b * CPAD + gv * 16, 16)] = vals + _pb

    sel_pos = []
    for k in range(8):
        def scan_body(i, carry):
            bv, bi = carry
            val = m_ref[pl.ds(i * 16, 16)]
            idx = i * 16 + lanes
            better = (val > bv) | ((val == bv) & (idx < bi))
            return (jnp.where(better, val, bv), jnp.where(better, idx, bi))

        bv, bi = pl.loop(
            0, (N_BEAMS * CPAD) // 16,
            init_carry=(jnp.full((16,), NEG, jnp.float32),
                        jnp.full((16,), BIG, jnp.int32)),
        )(scan_body)
        m = jnp.max(bv)
        pos = jnp.min(jnp.where(bv == m, bi, BIG))
        sel_pos.append(pos)
        plsc.store_scatter(m_ref, [jnp.full((16,), pos, jnp.int32)],
                           jnp.full((16,), NEG, jnp.float32), mask=lanes == 0)

    mvecs = []
    for k in range(8):
        pos = sel_pos[k]
        b_k = pos // CPAD
        g_k = pos % CPAD
        m0 = g_k * MPC
        pltpu.sync_copy(sh_micro.at[pl.ds(m0 * 64, MPC * 64)],
                        mg_ref.at[pl.ds(k * MPC * 64, MPC * 64)])
        pbv = plsc.load_gather(bp_ref, [jnp.full((16,), b_k, jnp.int32)])
        idx = k * MPC * 64 + lanes * 64 + jloc * 16 + (lb0 + b_k)
        mask = lanes < MPC
        vals = plsc.load_gather(mg_ref, [jnp.where(mask, idx, 0)])
        gbase = b_k * VOCAB + (m0 + lanes) * MW
        mvecs.append((jnp.where(mask, vals + pbv, NEG),
                      jnp.where(mask, gbase, BIG)))

    sel_micro = []
    for k in range(8):
        bv = jnp.full((16,), NEG, jnp.float32)
        bg = jnp.full((16,), BIG, jnp.int32)
        for mv, mg in mvecs:
            better = (mv > bv) | ((mv == bv) & (mg < bg))
            bv = jnp.where(better, mv, bv)
            bg = jnp.where(better, mg, bg)
        m = jnp.max(bv)
        gw = jnp.min(jnp.where(bv == m, bg, BIG))
        sel_micro.append(gw)
        mvecs = [(jnp.where(mg == gw, NEG, mv), mg) for mv, mg in mvecs]

    copies = []
    minfo = []
    for k in range(8):
        gw = sel_micro[k]
        b_m = gw // VOCAB
        v0 = pl.multiple_of(gw % VOCAB, MW)
        minfo.append((b_m, v0))
        copies.append(
            pltpu.async_copy(lp_hbm.at[pl.ds(v0, MW), :], tile_ref.at[k],
                             sem))
    for cp in copies:
        cp.wait()

    evecs = []
    for k in range(8):
        b_m, v0 = minfo[k]
        pbv = plsc.load_gather(bp_ref, [jnp.full((16,), b_m, jnp.int32)])
        lane_g = 64 * cid + 4 * sid + b_m
        mask = lanes < MW
        vals = plsc.load_gather(
            tile_ref, [jnp.full((16,), k, jnp.int32),
                       jnp.where(mask, lanes, 0),
                       jnp.full((16,), lane_g, jnp.int32)])
        g = b_m * VOCAB + v0 + lanes
        evecs.append((jnp.where(mask, vals + pbv, NEG),
                      jnp.where(mask, g, BIG)))

    winners = []
    for k in range(8):
        bv = jnp.full((16,), NEG, jnp.float32)
        bg = jnp.full((16,), BIG, jnp.int32)
        for ev, eg in evecs:
            better = (ev > bv) | ((ev == bv) & (eg < bg))
            bv = jnp.where(better, ev, bv)
            bg = jnp.where(better, eg, bg)
        m = jnp.max(bv)
        gi = jnp.min(jnp.where(bv == m, bg, BIG))
        winners.append((m, gi))
        evecs = [(jnp.where(eg == gi, NEG, ev), eg) for ev, eg in evecs]

    fvals = jnp.full((16,), NEG, jnp.float32)
    vocab_v = jnp.zeros((16,), jnp.int32)
    beam_v = jnp.zeros((16,), jnp.int32)
    for k in range(8):
        m, gi = winners[k]
        vocab_k = gi % VOCAB
        beam_k = gi // VOCAB
        mk = jnp.where(vocab_k == EOS_ID, KILL, m)
        fvals = jnp.where(lanes == k, mk, fvals)
        vocab_v = jnp.where(lanes == k, vocab_k, vocab_v)
        beam_v = jnp.where(lanes == k, beam_k, beam_v)

    out_f = jnp.zeros((16,), jnp.float32)
    out_i = jnp.zeros((16,), jnp.int32)
    for j in range(4):
        m = jnp.max(fvals)
        pos = jnp.min(jnp.where(fvals == m, lanes, BIG))
        onehot = lanes == pos
        vj = jnp.max(jnp.where(onehot, vocab_v, -1))
        bj = jnp.max(jnp.where(onehot, beam_v, -1))
        out_f = jnp.where(lanes == j, m, out_f)
        out_i = jnp.where(lanes == 4 + j, vj, out_i)
        out_i = jnp.where(lanes == 8 + j, bj, out_i)
        fvals = jnp.where(onehot, NEG, fvals)

    orow_ref[...] = jnp.where(lanes < 4, out_f,
                              plsc.bitcast(out_i, jnp.float32))
    pltpu.sync_copy(orow_ref, out_hbm.at[r])


def kernel(log_probs, best_prev):
    bp_pad = jnp.pad(best_prev, ((0, 0), (0, 16 - N_BEAMS)))
    out = _beam_topk(log_probs.T, bp_pad)
    cont = out[:, 0:4]
    vocab = lax.bitcast_convert_type(out[:, 4:8], jnp.int32)
    beam = lax.bitcast_convert_type(out[:, 8:12], jnp.int32)
    return cont, vocab, beam, vocab.reshape(-1)


# speedup vs baseline: 3.7421x; 1.0795x over previous
import functools

import jax
import jax.numpy as jnp
from jax import lax
from jax.experimental import pallas as pl
from jax.experimental.pallas import tpu as pltpu
from jax.experimental.pallas import tpu_sc as plsc

N_BEAMS = 4
EOS_ID = 2
KILL = -1000000000.0
VOCAB = 100000
ROWS = 32
LANES = 128
MW = 8
CW = 80
NMICRO = VOCAB // MW
NCHUNK = VOCAB // CW
CPAD = 1280
WIN = 6400
MPC = CW // MW
NEG = float("-inf")
BIG = 1 << 30

_mesh = plsc.VectorSubcoreMesh(
    core_axis_name="c", subcore_axis_name="s", num_cores=2, num_subcores=16
)


@functools.partial(
    pl.kernel,
    out_type=jax.ShapeDtypeStruct((ROWS, 16), jnp.float32),
    mesh=_mesh,
    compiler_params=pltpu.CompilerParams(needs_layout_passes=False),
    scratch_types=[
        pltpu.VMEM((CW, LANES), jnp.float32),
        pltpu.VMEM((CW, LANES), jnp.float32),
        pltpu.VMEM((2 * MPC * 4 * 16,), jnp.float32),
        pltpu.VMEM((4 * CW * 16,), jnp.float32),
        pltpu.VMEM((16 * CW * 16,), jnp.float32),
        pltpu.VMEM((N_BEAMS * CPAD,), jnp.float32),
        pltpu.VMEM((8 * MPC * 4 * 16,), jnp.float32),
        pltpu.VMEM((8, MW, LANES), jnp.float32),
        pltpu.VMEM((16,), jnp.float32),
        pltpu.VMEM((16,), jnp.float32),
        pltpu.VMEM_SHARED((NMICRO * 4 * 16 + 640,), jnp.float32),
        pltpu.VMEM_SHARED((4 * 16 * CW * 16,), jnp.float32),
        pltpu.SemaphoreType.DMA,
        pltpu.SemaphoreType.DMA,
        pltpu.SemaphoreType.DMA,
    ],
)
def _beam_topk(lp_hbm, bp_hbm, out_hbm, blk_a, blk_b, mstage, cstage,
               plane_ref, m_ref, mg_ref, tile_ref, bp_ref, orow_ref,
               sh_micro, sh_chunk, sem, sem_a, sem_b):
    cid = lax.axis_index("c")
    sid = lax.axis_index("s")
    r = cid * 16 + sid
    lanes = lax.iota(jnp.int32, 16)

    pltpu.sync_copy(bp_hbm.at[r], bp_ref)
    bp_vec = bp_ref[...]

    win0 = WIN * sid
    lastc = jnp.where(sid == 15, (4000 // CW) - 1, (WIN // CW) - 1)
    lane0 = 64 * cid

    @pl.loop(0, (4 * CW * 16) // 16)
    def _(i):
        cstage[pl.ds(i * 16, 16)] = jnp.full((16,), NEG, jnp.float32)

    def _src(ci):
        off = pl.multiple_of(win0 + CW * ci, 8)
        return lp_hbm.at[pl.ds(off, CW), :]

    def _reduce_block(buf, ci, half, l0):
        for jc in range(4):
            cvec = None
            for mi in range(MPC):
                v = buf[mi * MW, pl.ds(l0 + 16 * jc, 16)]
                for p in range(1, MW):
                    v = jnp.maximum(v, buf[mi * MW + p,
                                           pl.ds(l0 + 16 * jc, 16)])
                mstage[pl.ds(half * MPC * 64 + (mi * 4 + jc) * 16, 16)] = v
                cvec = v if cvec is None else jnp.maximum(cvec, v)
            cstage[pl.ds((jc * CW + ci) * 16, 16)] = cvec

    def _stage_a(l0):
        cl0 = jnp.minimum(0, lastc)
        pltpu.async_copy(_src(cl0), blk_a, sem_a)

        @pl.loop(0, (WIN // CW) // 2)
        def _(t):
            c0 = jnp.minimum(2 * t, lastc)
            c1 = jnp.minimum(2 * t + 1, lastc)
            pltpu.async_copy(_src(c1), blk_b, sem_b)
            pltpu.make_async_copy(_src(cl0), blk_a, sem_a).wait()
            _reduce_block(blk_a, c0, 0, l0)
            c2 = jnp.minimum(2 * t + 2, lastc)
            pltpu.async_copy(_src(c2), blk_a, sem_a)
            pltpu.make_async_copy(_src(cl0), blk_b, sem_b).wait()
            _reduce_block(blk_b, c1, 1, l0)
            m0 = (win0 + CW * c0) // MW
            pltpu.sync_copy(mstage, sh_micro.at[pl.ds(m0 * 64, 2 * MPC * 64)])

        pltpu.make_async_copy(_src(cl0), blk_a, sem_a).wait()

    @pl.when(cid == 0)
    def _():
        _stage_a(0)

    @pl.when(cid == 1)
    def _():
        _stage_a(64)

    for jc in range(4):
        pltpu.sync_copy(
            cstage.at[pl.ds(jc * CW * 16, CW * 16)],
            sh_chunk.at[pl.ds((jc * 16 + sid) * CW * 16, CW * 16)])

    plsc.subcore_barrier()

    jloc = sid // 4
    lb0 = 4 * (sid % 4)
    pltpu.sync_copy(sh_chunk.at[pl.ds(jloc * 16 * CW * 16, 16 * CW * 16)],
                    plane_ref)

    for b in range(N_BEAMS):
        pb = bp_vec[b]

        @pl.loop(0, CPAD // 16)
        def _(gv, _b=b, _pb=pb):
            idx = (gv * 16 + lanes) * 16 + (lb0 + _b)
            vals = plsc.load_gather(plane_ref, [idx])
            m_ref[pl.ds(_b * CPAD + gv * 16, 16)] = vals + _pb

    sel_pos = []
    for k in range(8):
        def scan_body(i, carry):
            bv, bi = carry
            val = m_ref[pl.ds(i * 16, 16)]
            idx = i * 16 + lanes
            better = (val > bv) | ((val == bv) & (idx < bi))
            return (jnp.where(better, val, bv), jnp.where(better, idx, bi))

        bv, bi = pl.loop(
            0, (N_BEAMS * CPAD) // 16,
            init_carry=(jnp.full((16,), NEG, jnp.float32),
                        jnp.full((16,), BIG, jnp.int32)),
        )(scan_body)
        m = jnp.max(bv)
        pos = jnp.min(jnp.where(bv == m, bi, BIG))
        sel_pos.append(pos)
        plsc.store_scatter(m_ref, [jnp.full((16,), pos, jnp.int32)],
                           jnp.full((16,), NEG, jnp.float32), mask=lanes == 0)

    mvecs = []
    for k in range(8):
        pos = sel_pos[k]
        b_k = pos // CPAD
        g_k = pos % CPAD
        m0 = g_k * MPC
        pltpu.sync_copy(sh_micro.at[pl.ds(m0 * 64, MPC * 64)],
                        mg_ref.at[pl.ds(k * MPC * 64, MPC * 64)])
        pbv = plsc.load_gather(bp_ref, [jnp.full((16,), b_k, jnp.int32)])
        idx = k * MPC * 64 + lanes * 64 + jloc * 16 + (lb0 + b_k)
        mask = lanes < MPC
        vals = plsc.load_gather(mg_ref, [jnp.where(mask, idx, 0)])
        gbase = b_k * VOCAB + (m0 + lanes) * MW
        mvecs.append((jnp.where(mask, vals + pbv, NEG),
                      jnp.where(mask, gbase, BIG)))

    sel_micro = []
    for k in range(8):
        bv = jnp.full((16,), NEG, jnp.float32)
        bg = jnp.full((16,), BIG, jnp.int32)
        for mv, mg in mvecs:
            better = (mv > bv) | ((mv == bv) & (mg < bg))
            bv = jnp.where(better, mv, bv)
            bg = jnp.where(better, mg, bg)
        m = jnp.max(bv)
        gw = jnp.min(jnp.where(bv == m, bg, BIG))
        sel_micro.append(gw)
        mvecs = [(jnp.where(mg == gw, NEG, mv), mg) for mv, mg in mvecs]

    copies = []
    minfo = []
    for k in range(8):
        gw = sel_micro[k]
        b_m = gw // VOCAB
        v0 = pl.multiple_of(gw % VOCAB, MW)
        minfo.append((b_m, v0))
        copies.append(
            pltpu.async_copy(lp_hbm.at[pl.ds(v0, MW), :], tile_ref.at[k],
                             sem))
    for cp in copies:
        cp.wait()

    evecs = []
    for k in range(8):
        b_m, v0 = minfo[k]
        pbv = plsc.load_gather(bp_ref, [jnp.full((16,), b_m, jnp.int32)])
        lane_g = 64 * cid + 4 * sid + b_m
        mask = lanes < MW
        vals = plsc.load_gather(
            tile_ref, [jnp.full((16,), k, jnp.int32),
                       jnp.where(mask, lanes, 0),
                       jnp.full((16,), lane_g, jnp.int32)])
        g = b_m * VOCAB + v0 + lanes
        evecs.append((jnp.where(mask, vals + pbv, NEG),
                      jnp.where(mask, g, BIG)))

    winners = []
    for k in range(8):
        bv = jnp.full((16,), NEG, jnp.float32)
        bg = jnp.full((16,), BIG, jnp.int32)
        for ev, eg in evecs:
            better = (ev > bv) | ((ev == bv) & (eg < bg))
            bv = jnp.where(better, ev, bv)
            bg = jnp.where(better, eg, bg)
        m = jnp.max(bv)
        gi = jnp.min(jnp.where(bv == m, bg, BIG))
        winners.append((m, gi))
        evecs = [(jnp.where(eg == gi, NEG, ev), eg) for ev, eg in evecs]

    fvals = jnp.full((16,), NEG, jnp.float32)
    vocab_v = jnp.zeros((16,), jnp.int32)
    beam_v = jnp.zeros((16,), jnp.int32)
    for k in range(8):
        m, gi = winners[k]
        vocab_k = gi % VOCAB
        beam_k = gi // VOCAB
        mk = jnp.where(vocab_k == EOS_ID, KILL, m)
        fvals = jnp.where(lanes == k, mk, fvals)
        vocab_v = jnp.where(lanes == k, vocab_k, vocab_v)
        beam_v = jnp.where(lanes == k, beam_k, beam_v)

    out_f = jnp.zeros((16,), jnp.float32)
    out_i = jnp.zeros((16,), jnp.int32)
    for j in range(4):
        m = jnp.max(fvals)
        pos = jnp.min(jnp.where(fvals == m, lanes, BIG))
        onehot = lanes == pos
        vj = jnp.max(jnp.where(onehot, vocab_v, -1))
        bj = jnp.max(jnp.where(onehot, beam_v, -1))
        out_f = jnp.where(lanes == j, m, out_f)
        out_i = jnp.where(lanes == 4 + j, vj, out_i)
        out_i = jnp.where(lanes == 8 + j, bj, out_i)
        fvals = jnp.where(onehot, NEG, fvals)

    orow_ref[...] = jnp.where(lanes < 4, out_f,
                              plsc.bitcast(out_i, jnp.float32))
    pltpu.sync_copy(orow_ref, out_hbm.at[r])


def kernel(log_probs, best_prev):
    bp_pad = jnp.pad(best_prev, ((0, 0), (0, 16 - N_BEAMS)))
    out = _beam_topk(log_probs.T, bp_pad)
    cont = out[:, 0:4]
    vocab = lax.bitcast_convert_type(out[:, 4:8], jnp.int32)
    beam = lax.bitcast_convert_type(out[:, 8:12], jnp.int32)
    return cont, vocab, beam, vocab.reshape(-1)


# speedup vs baseline: 4.3893x; 1.1729x over previous
import functools

import jax
import jax.numpy as jnp
from jax import lax
from jax.experimental import pallas as pl
from jax.experimental.pallas import tpu as pltpu
from jax.experimental.pallas import tpu_sc as plsc

N_BEAMS = 4
EOS_ID = 2
KILL = -1000000000.0
VOCAB = 100000
ROWS = 32
LANES = 128
MW = 8
CW = 80
NMICRO = VOCAB // MW
NCHUNK = VOCAB // CW
CPAD = 1280
WIN = 3200
WCH = WIN // CW
MPC = CW // MW
CT_N = 8 * CPAD * 16
MT_N = NMICRO * LANES + 2560
NEG = float("-inf")
BIG = 1 << 30

_mesh = plsc.VectorSubcoreMesh(
    core_axis_name="c", subcore_axis_name="s", num_cores=2, num_subcores=16
)


@functools.partial(
    pl.kernel,
    out_type=(jax.ShapeDtypeStruct((CT_N,), jnp.float32),
              jax.ShapeDtypeStruct((MT_N,), jnp.float32)),
    mesh=_mesh,
    compiler_params=pltpu.CompilerParams(needs_layout_passes=False),
    scratch_types=[
        pltpu.VMEM((CW, LANES), jnp.float32),
        pltpu.VMEM((CW, LANES), jnp.float32),
        pltpu.VMEM((2 * MPC * 8 * 16,), jnp.float32),
        pltpu.VMEM((8 * WCH * 16,), jnp.float32),
        pltpu.SemaphoreType.DMA,
        pltpu.SemaphoreType.DMA,
        pltpu.SemaphoreType.DMA,
    ],
)
def _reduce_k(lp_hbm, ct_hbm, mt_hbm, blk_a, blk_b, mstage, cstage,
              sem, sem_a, sem_b):
    cid = lax.axis_index("c")
    sid = lax.axis_index("s")
    w = cid * 16 + sid
    win0 = WIN * w
    lastc = jnp.where(w == 31, (800 // CW) - 1, WCH - 1)

    @pl.loop(0, (8 * WCH * 16) // 16)
    def _(i):
        cstage[pl.ds(i * 16, 16)] = jnp.full((16,), NEG, jnp.float32)

    def _src(ci):
        off = pl.multiple_of(win0 + CW * ci, 8)
        return lp_hbm.at[pl.ds(off, CW), :]

    def _reduce_block(buf, ci, half):
        for jb in range(8):
            cvec = None
            for mi in range(MPC):
                v = buf[mi * MW, pl.ds(16 * jb, 16)]
                for p in range(1, MW):
                    v = jnp.maximum(v, buf[mi * MW + p, pl.ds(16 * jb, 16)])
                mstage[pl.ds(half * MPC * 128 + (mi * 8 + jb) * 16, 16)] = v
                cvec = v if cvec is None else jnp.maximum(cvec, v)
            cstage[pl.ds((jb * WCH + ci) * 16, 16)] = cvec

    pltpu.async_copy(_src(jnp.minimum(0, lastc)), blk_a, sem_a)

    @pl.loop(0, WCH // 2)
    def _(t):
        c0 = jnp.minimum(2 * t, lastc)
        c1 = jnp.minimum(2 * t + 1, lastc)
        pltpu.async_copy(_src(c1), blk_b, sem_b)
        pltpu.make_async_copy(_src(c0), blk_a, sem_a).wait()
        _reduce_block(blk_a, c0, 0)
        c2 = jnp.minimum(2 * t + 2, lastc)
        pltpu.async_copy(_src(c2), blk_a, sem_a)
        pltpu.make_async_copy(_src(c1), blk_b, sem_b).wait()
        _reduce_block(blk_b, c1, 1)
        m0 = (win0 + CW * c0) // MW
        pltpu.sync_copy(mstage, mt_hbm.at[pl.ds(m0 * LANES, 2 * MPC * LANES)])

    pltpu.make_async_copy(_src(jnp.minimum(0, lastc)), blk_a, sem_a).wait()

    for jb in range(8):
        pltpu.sync_copy(
            cstage.at[pl.ds(jb * WCH * 16, WCH * 16)],
            ct_hbm.at[pl.ds((jb * CPAD + WCH * w) * 16, WCH * 16)])


@functools.partial(
    pl.kernel,
    out_type=jax.ShapeDtypeStruct((ROWS, 16), jnp.float32),
    mesh=_mesh,
    compiler_params=pltpu.CompilerParams(needs_layout_passes=False),
    scratch_types=[
        pltpu.VMEM((CPAD * 16,), jnp.float32),
        pltpu.VMEM((N_BEAMS * CPAD,), jnp.float32),
        pltpu.VMEM((8 * MPC * LANES,), jnp.float32),
        pltpu.VMEM((8, MW, LANES), jnp.float32),
        pltpu.VMEM((16,), jnp.float32),
        pltpu.VMEM((16,), jnp.float32),
        pltpu.SemaphoreType.DMA,
    ],
)
def _select_k(lp_hbm, bp_hbm, ct_hbm, mt_hbm, out_hbm, plane_ref, m_ref,
              mg_ref, tile_ref, bp_ref, orow_ref, sem):
    cid = lax.axis_index("c")
    sid = lax.axis_index("s")
    r = cid * 16 + sid
    lanes = lax.iota(jnp.int32, 16)

    pltpu.sync_copy(bp_hbm.at[r], bp_ref)
    bp_vec = bp_ref[...]

    jg = r // 4
    lb0 = 4 * (r % 4)
    pltpu.sync_copy(ct_hbm.at[pl.ds(jg * CPAD * 16, CPAD * 16)], plane_ref)

    for b in range(N_BEAMS):
        pb = bp_vec[b]

        @pl.loop(0, CPAD // 16)
        def _(gv, _b=b, _pb=pb):
            idx = (gv * 16 + lanes) * 16 + (lb0 + _b)
            vals = plsc.load_gather(plane_ref, [idx])
            m_ref[pl.ds(_b * CPAD + gv * 16, 16)] = vals + _pb

    sel_pos = []
    for k in range(8):
        def scan_body(i, carry):
            bv, bi = carry
            val = m_ref[pl.ds(i * 16, 16)]
            idx = i * 16 + lanes
            better = (val > bv) | ((val == bv) & (idx < bi))
            return (jnp.where(better, val, bv), jnp.where(better, idx, bi))

        bv, bi = pl.loop(
            0, (N_BEAMS * CPAD) // 16,
            init_carry=(jnp.full((16,), NEG, jnp.float32),
                        jnp.full((16,), BIG, jnp.int32)),
        )(scan_body)
        m = jnp.max(bv)
        pos = jnp.min(jnp.where(bv == m, bi, BIG))
        sel_pos.append(pos)
        plsc.store_scatter(m_ref, [jnp.full((16,), pos, jnp.int32)],
                           jnp.full((16,), NEG, jnp.float32), mask=lanes == 0)

    mvecs = []
    for k in range(8):
        pos = sel_pos[k]
        b_k = pos // CPAD
        g_k = pos % CPAD
        m0 = g_k * MPC
        pltpu.sync_copy(mt_hbm.at[pl.ds(m0 * LANES, MPC * LANES)],
                        mg_ref.at[pl.ds(k * MPC * LANES, MPC * LANES)])
        pbv = plsc.load_gather(bp_ref, [jnp.full((16,), b_k, jnp.int32)])
        idx = k * MPC * LANES + lanes * LANES + jg * 16 + (lb0 + b_k)
        mask = lanes < MPC
        vals = plsc.load_gather(mg_ref, [jnp.where(mask, idx, 0)])
        gbase = b_k * VOCAB + (m0 + lanes) * MW
        mvecs.append((jnp.where(mask, vals + pbv, NEG),
                      jnp.where(mask, gbase, BIG)))

    sel_micro = []
    for k in range(8):
        bv = jnp.full((16,), NEG, jnp.float32)
        bg = jnp.full((16,), BIG, jnp.int32)
        for mv, mg in mvecs:
            better = (mv > bv) | ((mv == bv) & (mg < bg))
            bv = jnp.where(better, mv, bv)
            bg = jnp.where(better, mg, bg)
        m = jnp.max(bv)
        gw = jnp.min(jnp.where(bv == m, bg, BIG))
        sel_micro.append(gw)
        mvecs = [(jnp.where(mg == gw, NEG, mv), mg) for mv, mg in mvecs]

    copies = []
    minfo = []
    for k in range(8):
        gw = sel_micro[k]
        b_m = gw // VOCAB
        v0 = pl.multiple_of(gw % VOCAB, MW)
        minfo.append((b_m, v0))
        copies.append(
            pltpu.async_copy(lp_hbm.at[pl.ds(v0, MW), :], tile_ref.at[k],
                             sem))
    for cp in copies:
        cp.wait()

    evecs = []
    for k in range(8):
        b_m, v0 = minfo[k]
        pbv = plsc.load_gather(bp_ref, [jnp.full((16,), b_m, jnp.int32)])
        lane_g = 4 * r + b_m
        mask = lanes < MW
        vals = plsc.load_gather(
            tile_ref, [jnp.full((16,), k, jnp.int32),
                       jnp.where(mask, lanes, 0),
                       jnp.full((16,), lane_g, jnp.int32)])
        g = b_m * VOCAB + v0 + lanes
        evecs.append((jnp.where(mask, vals + pbv, NEG),
                      jnp.where(mask, g, BIG)))

    winners = []
    for k in range(8):
        bv = jnp.full((16,), NEG, jnp.float32)
        bg = jnp.full((16,), BIG, jnp.int32)
        for ev, eg in evecs:
            better = (ev > bv) | ((ev == bv) & (eg < bg))
            bv = jnp.where(better, ev, bv)
            bg = jnp.where(better, eg, bg)
        m = jnp.max(bv)
        gi = jnp.min(jnp.where(bv == m, bg, BIG))
        winners.append((m, gi))
        evecs = [(jnp.where(eg == gi, NEG, ev), eg) for ev, eg in evecs]

    fvals = jnp.full((16,), NEG, jnp.float32)
    vocab_v = jnp.zeros((16,), jnp.int32)
    beam_v = jnp.zeros((16,), jnp.int32)
    for k in range(8):
        m, gi = winners[k]
        vocab_k = gi % VOCAB
        beam_k = gi // VOCAB
        mk = jnp.where(vocab_k == EOS_ID, KILL, m)
        fvals = jnp.where(lanes == k, mk, fvals)
        vocab_v = jnp.where(lanes == k, vocab_k, vocab_v)
        beam_v = jnp.where(lanes == k, beam_k, beam_v)

    out_f = jnp.zeros((16,), jnp.float32)
    out_i = jnp.zeros((16,), jnp.int32)
    for j in range(4):
        m = jnp.max(fvals)
        pos = jnp.min(jnp.where(fvals == m, lanes, BIG))
        onehot = lanes == pos
        vj = jnp.max(jnp.where(onehot, vocab_v, -1))
        bj = jnp.max(jnp.where(onehot, beam_v, -1))
        out_f = jnp.where(lanes == j, m, out_f)
        out_i = jnp.where(lanes == 4 + j, vj, out_i)
        out_i = jnp.where(lanes == 8 + j, bj, out_i)
        fvals = jnp.where(onehot, NEG, fvals)

    orow_ref[...] = jnp.where(lanes < 4, out_f,
                              plsc.bitcast(out_i, jnp.float32))
    pltpu.sync_copy(orow_ref, out_hbm.at[r])


def kernel(log_probs, best_prev):
    bp_pad = jnp.pad(best_prev, ((0, 0), (0, 16 - N_BEAMS)))
    lpt = log_probs.T
    ct, mt = _reduce_k(lpt)
    out = _select_k(lpt, bp_pad, ct, mt)
    cont = out[:, 0:4]
    vocab = lax.bitcast_convert_type(out[:, 4:8], jnp.int32)
    beam = lax.bitcast_convert_type(out[:, 8:12], jnp.int32)
    return cont, vocab, beam, vocab.reshape(-1)


# speedup vs baseline: 4.5359x; 1.0334x over previous
import functools

import jax
import jax.numpy as jnp
from jax import lax
from jax.experimental import pallas as pl
from jax.experimental.pallas import tpu as pltpu
from jax.experimental.pallas import tpu_sc as plsc

N_BEAMS = 4
EOS_ID = 2
KILL = -1000000000.0
VOCAB = 100000
ROWS = 32
LANES = 128
MW = 8
CW = 80
NMICRO = VOCAB // MW
NCHUNK = VOCAB // CW
CPAD = 1280
WIN = 3200
WCH = WIN // CW
MPC = CW // MW
CT_N = 8 * CPAD * 16
MT_N = NMICRO * LANES + 2560
NEG = float("-inf")
BIG = 1 << 30

_mesh = plsc.VectorSubcoreMesh(
    core_axis_name="c", subcore_axis_name="s", num_cores=2, num_subcores=16
)


@functools.partial(
    pl.kernel,
    out_type=(jax.ShapeDtypeStruct((CT_N,), jnp.float32),
              jax.ShapeDtypeStruct((MT_N,), jnp.float32)),
    mesh=_mesh,
    compiler_params=pltpu.CompilerParams(needs_layout_passes=False),
    scratch_types=[
        pltpu.VMEM((CW, LANES), jnp.float32),
        pltpu.VMEM((CW, LANES), jnp.float32),
        pltpu.VMEM((2 * MPC * 8 * 16,), jnp.float32),
        pltpu.VMEM((8 * WCH * 16,), jnp.float32),
        pltpu.SemaphoreType.DMA,
        pltpu.SemaphoreType.DMA,
        pltpu.SemaphoreType.DMA,
    ],
)
def _reduce_k(lp_hbm, ct_hbm, mt_hbm, blk_a, blk_b, mstage, cstage,
              sem, sem_a, sem_b):
    cid = lax.axis_index("c")
    sid = lax.axis_index("s")
    w = cid * 16 + sid
    win0 = WIN * w
    lastc = jnp.where(w == 31, (800 // CW) - 1, WCH - 1)

    @pl.loop(0, (8 * WCH * 16) // 16)
    def _(i):
        cstage[pl.ds(i * 16, 16)] = jnp.full((16,), NEG, jnp.float32)

    def _src(ci):
        off = pl.multiple_of(win0 + CW * ci, 8)
        return lp_hbm.at[pl.ds(off, CW), :]

    def _reduce_block(buf, ci, half):
        for jb in range(8):
            cvec = None
            for mi in range(MPC):
                v = buf[mi * MW, pl.ds(16 * jb, 16)]
                for p in range(1, MW):
                    v = jnp.maximum(v, buf[mi * MW + p, pl.ds(16 * jb, 16)])
                mstage[pl.ds(half * MPC * 128 + (mi * 8 + jb) * 16, 16)] = v
                cvec = v if cvec is None else jnp.maximum(cvec, v)
            cstage[pl.ds((jb * WCH + ci) * 16, 16)] = cvec

    pltpu.async_copy(_src(jnp.minimum(0, lastc)), blk_a, sem_a)

    @pl.loop(0, WCH // 2)
    def _(t):
        c0 = jnp.minimum(2 * t, lastc)
        c1 = jnp.minimum(2 * t + 1, lastc)
        pltpu.async_copy(_src(c1), blk_b, sem_b)
        pltpu.make_async_copy(_src(c0), blk_a, sem_a).wait()
        _reduce_block(blk_a, c0, 0)
        c2 = jnp.minimum(2 * t + 2, lastc)
        pltpu.async_copy(_src(c2), blk_a, sem_a)
        pltpu.make_async_copy(_src(c1), blk_b, sem_b).wait()
        _reduce_block(blk_b, c1, 1)
        m0 = (win0 + CW * c0) // MW
        pltpu.sync_copy(mstage, mt_hbm.at[pl.ds(m0 * LANES, 2 * MPC * LANES)])

    pltpu.make_async_copy(_src(jnp.minimum(0, lastc)), blk_a, sem_a).wait()

    for jb in range(8):
        pltpu.sync_copy(
            cstage.at[pl.ds(jb * WCH * 16, WCH * 16)],
            ct_hbm.at[pl.ds((jb * CPAD + WCH * w) * 16, WCH * 16)])


@functools.partial(
    pl.kernel,
    out_type=jax.ShapeDtypeStruct((ROWS, 16), jnp.float32),
    mesh=_mesh,
    compiler_params=pltpu.CompilerParams(needs_layout_passes=False),
    scratch_types=[
        pltpu.VMEM((CPAD * 16,), jnp.float32),
        pltpu.VMEM((N_BEAMS * CPAD,), jnp.float32),
        pltpu.VMEM((8 * MPC * LANES,), jnp.float32),
        pltpu.VMEM((8, MW, LANES), jnp.float32),
        pltpu.VMEM((16,), jnp.float32),
        pltpu.VMEM((16,), jnp.float32),
        pltpu.SemaphoreType.DMA,
    ],
)
def _select_k(lp_hbm, bp_hbm, ct_hbm, mt_hbm, out_hbm, plane_ref, m_ref,
              mg_ref, tile_ref, bp_ref, orow_ref, sem):
    cid = lax.axis_index("c")
    sid = lax.axis_index("s")
    r = cid * 16 + sid
    lanes = lax.iota(jnp.int32, 16)

    pltpu.sync_copy(bp_hbm.at[r], bp_ref)
    bp_vec = bp_ref[...]

    jg = r // 4
    lb0 = 4 * (r % 4)
    pltpu.sync_copy(ct_hbm.at[pl.ds(jg * CPAD * 16, CPAD * 16)], plane_ref)

    for b in range(N_BEAMS):
        pb = bp_vec[b]

        @pl.loop(0, CPAD // 16, unroll=4)
        def _(gv, _b=b, _pb=pb):
            idx = (gv * 16 + lanes) * 16 + (lb0 + _b)
            vals = plsc.load_gather(plane_ref, [idx])
            m_ref[pl.ds(_b * CPAD + gv * 16, 16)] = vals + _pb

    sel_pos = []
    for k in range(8):
        def scan_body(i, carry):
            bv, bi = carry
            val = m_ref[pl.ds(i * 16, 16)]
            idx = i * 16 + lanes
            better = (val > bv) | ((val == bv) & (idx < bi))
            return (jnp.where(better, val, bv), jnp.where(better, idx, bi))

        bv, bi = pl.loop(
            0, (N_BEAMS * CPAD) // 16, unroll=4,
            init_carry=(jnp.full((16,), NEG, jnp.float32),
                        jnp.full((16,), BIG, jnp.int32)),
        )(scan_body)
        m = jnp.max(bv)
        pos = jnp.min(jnp.where(bv == m, bi, BIG))
        sel_pos.append(pos)
        plsc.store_scatter(m_ref, [jnp.full((16,), pos, jnp.int32)],
                           jnp.full((16,), NEG, jnp.float32), mask=lanes == 0)

    mvecs = []
    for k in range(8):
        pos = sel_pos[k]
        b_k = pos // CPAD
        g_k = pos % CPAD
        m0 = g_k * MPC
        pltpu.sync_copy(mt_hbm.at[pl.ds(m0 * LANES, MPC * LANES)],
                        mg_ref.at[pl.ds(k * MPC * LANES, MPC * LANES)])
        pbv = plsc.load_gather(bp_ref, [jnp.full((16,), b_k, jnp.int32)])
        idx = k * MPC * LANES + lanes * LANES + jg * 16 + (lb0 + b_k)
        mask = lanes < MPC
        vals = plsc.load_gather(mg_ref, [jnp.where(mask, idx, 0)])
        gbase = b_k * VOCAB + (m0 + lanes) * MW
        mvecs.append((jnp.where(mask, vals + pbv, NEG),
                      jnp.where(mask, gbase, BIG)))

    sel_micro = []
    for k in range(8):
        bv = jnp.full((16,), NEG, jnp.float32)
        bg = jnp.full((16,), BIG, jnp.int32)
        for mv, mg in mvecs:
            better = (mv > bv) | ((mv == bv) & (mg < bg))
            bv = jnp.where(better, mv, bv)
            bg = jnp.where(better, mg, bg)
        m = jnp.max(bv)
        gw = jnp.min(jnp.where(bv == m, bg, BIG))
        sel_micro.append(gw)
        mvecs = [(jnp.where(mg == gw, NEG, mv), mg) for mv, mg in mvecs]

    copies = []
    minfo = []
    for k in range(8):
        gw = sel_micro[k]
        b_m = gw // VOCAB
        v0 = pl.multiple_of(gw % VOCAB, MW)
        minfo.append((b_m, v0))
        copies.append(
            pltpu.async_copy(lp_hbm.at[pl.ds(v0, MW), :], tile_ref.at[k],
                             sem))
    for cp in copies:
        cp.wait()

    evecs = []
    for k in range(8):
        b_m, v0 = minfo[k]
        pbv = plsc.load_gather(bp_ref, [jnp.full((16,), b_m, jnp.int32)])
        lane_g = 4 * r + b_m
        mask = lanes < MW
        vals = plsc.load_gather(
            tile_ref, [jnp.full((16,), k, jnp.int32),
                       jnp.where(mask, lanes, 0),
                       jnp.full((16,), lane_g, jnp.int32)])
        g = b_m * VOCAB + v0 + lanes
        evecs.append((jnp.where(mask, vals + pbv, NEG),
                      jnp.where(mask, g, BIG)))

    winners = []
    for k in range(8):
        bv = jnp.full((16,), NEG, jnp.float32)
        bg = jnp.full((16,), BIG, jnp.int32)
        for ev, eg in evecs:
            better = (ev > bv) | ((ev == bv) & (eg < bg))
            bv = jnp.where(better, ev, bv)
            bg = jnp.where(better, eg, bg)
        m = jnp.max(bv)
        gi = jnp.min(jnp.where(bv == m, bg, BIG))
        winners.append((m, gi))
        evecs = [(jnp.where(eg == gi, NEG, ev), eg) for ev, eg in evecs]

    fvals = jnp.full((16,), NEG, jnp.float32)
    vocab_v = jnp.zeros((16,), jnp.int32)
    beam_v = jnp.zeros((16,), jnp.int32)
    for k in range(8):
        m, gi = winners[k]
        vocab_k = gi % VOCAB
        beam_k = gi // VOCAB
        mk = jnp.where(vocab_k == EOS_ID, KILL, m)
        fvals = jnp.where(lanes == k, mk, fvals)
        vocab_v = jnp.where(lanes == k, vocab_k, vocab_v)
        beam_v = jnp.where(lanes == k, beam_k, beam_v)

    out_f = jnp.zeros((16,), jnp.float32)
    out_i = jnp.zeros((16,), jnp.int32)
    for j in range(4):
        m = jnp.max(fvals)
        pos = jnp.min(jnp.where(fvals == m, lanes, BIG))
        onehot = lanes == pos
        vj = jnp.max(jnp.where(onehot, vocab_v, -1))
        bj = jnp.max(jnp.where(onehot, beam_v, -1))
        out_f = jnp.where(lanes == j, m, out_f)
        out_i = jnp.where(lanes == 4 + j, vj, out_i)
        out_i = jnp.where(lanes == 8 + j, bj, out_i)
        fvals = jnp.where(onehot, NEG, fvals)

    orow_ref[...] = jnp.where(lanes < 4, out_f,
                              plsc.bitcast(out_i, jnp.float32))
    pltpu.sync_copy(orow_ref, out_hbm.at[r])


def kernel(log_probs, best_prev):
    bp_pad = jnp.pad(best_prev, ((0, 0), (0, 16 - N_BEAMS)))
    lpt = log_probs.T
    ct, mt = _reduce_k(lpt)
    out = _select_k(lpt, bp_pad, ct, mt)
    cont = out[:, 0:4]
    vocab = lax.bitcast_convert_type(out[:, 4:8], jnp.int32)
    beam = lax.bitcast_convert_type(out[:, 8:12], jnp.int32)
    return cont, vocab, beam, vocab.reshape(-1)


# speedup vs baseline: 4.5438x; 1.0017x over previous
import functools

import jax
import jax.numpy as jnp
from jax import lax
from jax.experimental import pallas as pl
from jax.experimental.pallas import tpu as pltpu
from jax.experimental.pallas import tpu_sc as plsc

N_BEAMS = 4
EOS_ID = 2
KILL = -1000000000.0
VOCAB = 100000
ROWS = 32
LANES = 128
MW = 8
CW = 80
NMICRO = VOCAB // MW
NCHUNK = VOCAB // CW
CPAD = 1280
WIN = 3200
WCH = WIN // CW
MPC = CW // MW
CT_N = 8 * CPAD * 16
MT_N = NMICRO * LANES + 2560
NEG = float("-inf")
BIG = 1 << 30

_mesh = plsc.VectorSubcoreMesh(
    core_axis_name="c", subcore_axis_name="s", num_cores=2, num_subcores=16
)


@functools.partial(
    pl.kernel,
    out_type=(jax.ShapeDtypeStruct((CT_N,), jnp.float32),
              jax.ShapeDtypeStruct((MT_N,), jnp.float32)),
    mesh=_mesh,
    compiler_params=pltpu.CompilerParams(needs_layout_passes=False),
    scratch_types=[
        pltpu.VMEM((CW, LANES), jnp.float32),
        pltpu.VMEM((CW, LANES), jnp.float32),
        pltpu.VMEM((2 * MPC * 8 * 16,), jnp.float32),
        pltpu.VMEM((8 * WCH * 16,), jnp.float32),
        pltpu.SemaphoreType.DMA,
        pltpu.SemaphoreType.DMA,
        pltpu.SemaphoreType.DMA,
    ],
)
def _reduce_k(lp_hbm, ct_hbm, mt_hbm, blk_a, blk_b, mstage, cstage,
              sem, sem_a, sem_b):
    cid = lax.axis_index("c")
    sid = lax.axis_index("s")
    w = cid * 16 + sid
    win0 = WIN * w
    lastc = jnp.where(w == 31, (800 // CW) - 1, WCH - 1)

    @pl.loop(0, (8 * WCH * 16) // 16)
    def _(i):
        cstage[pl.ds(i * 16, 16)] = jnp.full((16,), NEG, jnp.float32)

    def _src(ci):
        off = pl.multiple_of(win0 + CW * ci, 8)
        return lp_hbm.at[pl.ds(off, CW), :]

    def _reduce_block(buf, ci, half):
        for jb in range(8):
            cvec = None
            for mi in range(MPC):
                v = buf[mi * MW, pl.ds(16 * jb, 16)]
                for p in range(1, MW):
                    v = jnp.maximum(v, buf[mi * MW + p, pl.ds(16 * jb, 16)])
                mstage[pl.ds(half * MPC * 128 + (mi * 8 + jb) * 16, 16)] = v
                cvec = v if cvec is None else jnp.maximum(cvec, v)
            cstage[pl.ds((jb * WCH + ci) * 16, 16)] = cvec

    pltpu.async_copy(_src(jnp.minimum(0, lastc)), blk_a, sem_a)
    pltpu.async_copy(mstage, mt_hbm.at[pl.ds(MT_N - 2560, 2 * MPC * LANES)],
                     sem)

    @pl.loop(0, WCH // 2)
    def _(t):
        pltpu.make_async_copy(
            mstage, mt_hbm.at[pl.ds(MT_N - 2560, 2 * MPC * LANES)], sem).wait()
        c0 = jnp.minimum(2 * t, lastc)
        c1 = jnp.minimum(2 * t + 1, lastc)
        pltpu.async_copy(_src(c1), blk_b, sem_b)
        pltpu.make_async_copy(_src(c0), blk_a, sem_a).wait()
        _reduce_block(blk_a, c0, 0)
        c2 = jnp.minimum(2 * t + 2, lastc)
        pltpu.async_copy(_src(c2), blk_a, sem_a)
        pltpu.make_async_copy(_src(c1), blk_b, sem_b).wait()
        _reduce_block(blk_b, c1, 1)
        m0 = (win0 + CW * c0) // MW
        pltpu.async_copy(mstage,
                         mt_hbm.at[pl.ds(m0 * LANES, 2 * MPC * LANES)], sem)

    pltpu.make_async_copy(
        mstage, mt_hbm.at[pl.ds(MT_N - 2560, 2 * MPC * LANES)], sem).wait()
    pltpu.make_async_copy(_src(jnp.minimum(0, lastc)), blk_a, sem_a).wait()

    for jb in range(8):
        pltpu.sync_copy(
            cstage.at[pl.ds(jb * WCH * 16, WCH * 16)],
            ct_hbm.at[pl.ds((jb * CPAD + WCH * w) * 16, WCH * 16)])


@functools.partial(
    pl.kernel,
    out_type=jax.ShapeDtypeStruct((ROWS, 16), jnp.float32),
    mesh=_mesh,
    compiler_params=pltpu.CompilerParams(needs_layout_passes=False),
    scratch_types=[
        pltpu.VMEM((CPAD * 16,), jnp.float32),
        pltpu.VMEM((N_BEAMS * CPAD,), jnp.float32),
        pltpu.VMEM((8 * MPC * LANES,), jnp.float32),
        pltpu.VMEM((8, MW, LANES), jnp.float32),
        pltpu.VMEM((16,), jnp.float32),
        pltpu.VMEM((16,), jnp.float32),
        pltpu.SemaphoreType.DMA,
    ],
)
def _select_k(lp_hbm, bp_hbm, ct_hbm, mt_hbm, out_hbm, plane_ref, m_ref,
              mg_ref, tile_ref, bp_ref, orow_ref, sem):
    cid = lax.axis_index("c")
    sid = lax.axis_index("s")
    r = cid * 16 + sid
    lanes = lax.iota(jnp.int32, 16)

    pltpu.sync_copy(bp_hbm.at[r], bp_ref)
    bp_vec = bp_ref[...]

    jg = r // 4
    lb0 = 4 * (r % 4)
    pltpu.sync_copy(ct_hbm.at[pl.ds(jg * CPAD * 16, CPAD * 16)], plane_ref)

    for b in range(N_BEAMS):
        pb = bp_vec[b]

        @pl.loop(0, CPAD // 16, unroll=4)
        def _(gv, _b=b, _pb=pb):
            idx = (gv * 16 + lanes) * 16 + (lb0 + _b)
            vals = plsc.load_gather(plane_ref, [idx])
            m_ref[pl.ds(_b * CPAD + gv * 16, 16)] = vals + _pb

    sel_pos = []
    for k in range(8):
        def scan_body(i, carry):
            bv, bi = carry
            val = m_ref[pl.ds(i * 16, 16)]
            idx = i * 16 + lanes
            better = (val > bv) | ((val == bv) & (idx < bi))
            return (jnp.where(better, val, bv), jnp.where(better, idx, bi))

        bv, bi = pl.loop(
            0, (N_BEAMS * CPAD) // 16, unroll=4,
            init_carry=(jnp.full((16,), NEG, jnp.float32),
                        jnp.full((16,), BIG, jnp.int32)),
        )(scan_body)
        m = jnp.max(bv)
        pos = jnp.min(jnp.where(bv == m, bi, BIG))
        sel_pos.append(pos)
        plsc.store_scatter(m_ref, [jnp.full((16,), pos, jnp.int32)],
                           jnp.full((16,), NEG, jnp.float32), mask=lanes == 0)

    mvecs = []
    for k in range(8):
        pos = sel_pos[k]
        b_k = pos // CPAD
        g_k = pos % CPAD
        m0 = g_k * MPC
        pltpu.sync_copy(mt_hbm.at[pl.ds(m0 * LANES, MPC * LANES)],
                        mg_ref.at[pl.ds(k * MPC * LANES, MPC * LANES)])
        pbv = plsc.load_gather(bp_ref, [jnp.full((16,), b_k, jnp.int32)])
        idx = k * MPC * LANES + lanes * LANES + jg * 16 + (lb0 + b_k)
        mask = lanes < MPC
        vals = plsc.load_gather(mg_ref, [jnp.where(mask, idx, 0)])
        gbase = b_k * VOCAB + (m0 + lanes) * MW
        mvecs.append((jnp.where(mask, vals + pbv, NEG),
                      jnp.where(mask, gbase, BIG)))

    sel_micro = []
    for k in range(8):
        bv = jnp.full((16,), NEG, jnp.float32)
        bg = jnp.full((16,), BIG, jnp.int32)
        for mv, mg in mvecs:
            better = (mv > bv) | ((mv == bv) & (mg < bg))
            bv = jnp.where(better, mv, bv)
            bg = jnp.where(better, mg, bg)
        m = jnp.max(bv)
        gw = jnp.min(jnp.where(bv == m, bg, BIG))
        sel_micro.append(gw)
        mvecs = [(jnp.where(mg == gw, NEG, mv), mg) for mv, mg in mvecs]

    copies = []
    minfo = []
    for k in range(8):
        gw = sel_micro[k]
        b_m = gw // VOCAB
        v0 = pl.multiple_of(gw % VOCAB, MW)
        minfo.append((b_m, v0))
        copies.append(
            pltpu.async_copy(lp_hbm.at[pl.ds(v0, MW), :], tile_ref.at[k],
                             sem))
    for cp in copies:
        cp.wait()

    evecs = []
    for k in range(8):
        b_m, v0 = minfo[k]
        pbv = plsc.load_gather(bp_ref, [jnp.full((16,), b_m, jnp.int32)])
        lane_g = 4 * r + b_m
        mask = lanes < MW
        vals = plsc.load_gather(
            tile_ref, [jnp.full((16,), k, jnp.int32),
                       jnp.where(mask, lanes, 0),
                       jnp.full((16,), lane_g, jnp.int32)])
        g = b_m * VOCAB + v0 + lanes
        evecs.append((jnp.where(mask, vals + pbv, NEG),
                      jnp.where(mask, g, BIG)))

    winners = []
    for k in range(8):
        bv = jnp.full((16,), NEG, jnp.float32)
        bg = jnp.full((16,), BIG, jnp.int32)
        for ev, eg in evecs:
            better = (ev > bv) | ((ev == bv) & (eg < bg))
            bv = jnp.where(better, ev, bv)
            bg = jnp.where(better, eg, bg)
        m = jnp.max(bv)
        gi = jnp.min(jnp.where(bv == m, bg, BIG))
        winners.append((m, gi))
        evecs = [(jnp.where(eg == gi, NEG, ev), eg) for ev, eg in evecs]

    fvals = jnp.full((16,), NEG, jnp.float32)
    vocab_v = jnp.zeros((16,), jnp.int32)
    beam_v = jnp.zeros((16,), jnp.int32)
    for k in range(8):
        m, gi = winners[k]
        vocab_k = gi % VOCAB
        beam_k = gi // VOCAB
        mk = jnp.where(vocab_k == EOS_ID, KILL, m)
        fvals = jnp.where(lanes == k, mk, fvals)
        vocab_v = jnp.where(lanes == k, vocab_k, vocab_v)
        beam_v = jnp.where(lanes == k, beam_k, beam_v)

    out_f = jnp.zeros((16,), jnp.float32)
    out_i = jnp.zeros((16,), jnp.int32)
    for j in range(4):
        m = jnp.max(fvals)
        pos = jnp.min(jnp.where(fvals == m, lanes, BIG))
        onehot = lanes == pos
        vj = jnp.max(jnp.where(onehot, vocab_v, -1))
        bj = jnp.max(jnp.where(onehot, beam_v, -1))
        out_f = jnp.where(lanes == j, m, out_f)
        out_i = jnp.where(lanes == 4 + j, vj, out_i)
        out_i = jnp.where(lanes == 8 + j, bj, out_i)
        fvals = jnp.where(onehot, NEG, fvals)

    orow_ref[...] = jnp.where(lanes < 4, out_f,
                              plsc.bitcast(out_i, jnp.float32))
    pltpu.sync_copy(orow_ref, out_hbm.at[r])


def kernel(log_probs, best_prev):
    bp_pad = jnp.pad(best_prev, ((0, 0), (0, 16 - N_BEAMS)))
    lpt = log_probs.T
    ct, mt = _reduce_k(lpt)
    out = _select_k(lpt, bp_pad, ct, mt)
    cont = out[:, 0:4]
    vocab = lax.bitcast_convert_type(out[:, 4:8], jnp.int32)
    beam = lax.bitcast_convert_type(out[:, 8:12], jnp.int32)
    return cont, vocab, beam, vocab.reshape(-1)
